# baseline (speedup 1.0000x reference)
import functools

import jax
import jax.numpy as jnp
from jax import lax
from jax.experimental import pallas as pl
from jax.experimental.pallas import tpu as pltpu
from jax.experimental.pallas import tpu_sc as plsc

D = 256
KSUB = D // 16
BLK = 128

_NEG_INF = float("-inf")


def _sget(ref, idx):
    return ref[pl.ds(idx, 16)][0]


_BSTEPS = (64, 32, 16, 8, 4, 2, 1)


def _segids16(csr_v, cb, iv, e0, e1):
    lo = jnp.zeros((16,), jnp.int32)
    for step in _BSTEPS:
        cand = lo + step
        v = plsc.load_gather(csr_v, [cb + cand])
        lo = jnp.where(v <= iv, cand, lo)
    return jnp.where((iv >= e0) & (iv < e1), lo, BLK)


def _scores_tc(x_map, w_att):
    n, d = x_map.shape
    blk = 1024

    def body(xm, w, o):
        o[...] = lax.dot_general(
            xm[...], w[...], (((1,), (0,)), ((), ())),
            preferred_element_type=jnp.float32)

    return pl.pallas_call(
        body,
        grid=(n // blk,),
        in_specs=[
            pl.BlockSpec((blk, d), lambda i: (i, 0)),
            pl.BlockSpec((d, 1), lambda i: (0, 0)),
        ],
        out_specs=pl.BlockSpec((blk, 1), lambda i: (i, 0)),
        out_shape=jax.ShapeDtypeStruct((n, 1), jnp.float32),
    )(x_map, w_att.reshape(d, 1))


def _fuse_tc(x_3d, x_pool, W_fuse, b_fuse):
    n, d = x_3d.shape
    blk = 1024
    w1 = W_fuse[:d]
    w2 = W_fuse[d:]
    b2 = b_fuse.reshape(1, d)

    def body(a, p, wa, wb, b, o):
        acc = lax.dot_general(a[...], wa[...], (((1,), (0,)), ((), ())),
                              preferred_element_type=jnp.float32)
        acc += lax.dot_general(p[...], wb[...], (((1,), (0,)), ((), ())),
                               preferred_element_type=jnp.float32)
        o[...] = acc + b[...]

    return pl.pallas_call(
        body,
        grid=(n // blk,),
        in_specs=[
            pl.BlockSpec((blk, d), lambda i: (i, 0)),
            pl.BlockSpec((blk, d), lambda i: (i, 0)),
            pl.BlockSpec((d, d), lambda i: (0, 0)),
            pl.BlockSpec((d, d), lambda i: (0, 0)),
            pl.BlockSpec((1, d), lambda i: (0, 0)),
        ],
        out_specs=pl.BlockSpec((blk, d), lambda i: (i, 0)),
        out_shape=jax.ShapeDtypeStruct((n, d), jnp.float32),
    )(x_3d, x_pool, w1, w2, b2)


def _atomic_pool_sc(mod_x, fmi, acsr, n_seg):
    info = plsc.get_sparse_core_info()
    nc, ns = info.num_cores, info.num_subcores
    nw = nc * ns
    segw = n_seg // nw
    nblk = segw // BLK
    csr_len = segw + 24
    WIN = 8

    fmi_pad = jnp.pad(fmi, (0, (WIN + 1) * BLK))
    csr_pad = jnp.pad(acsr, (0, 32), mode="edge")

    @functools.partial(
        pl.kernel,
        mesh=plsc.VectorSubcoreMesh(core_axis_name="c", subcore_axis_name="s"),
        out_type=jax.ShapeDtypeStruct((n_seg + 128, D), jnp.float32),
        compiler_params=pltpu.CompilerParams(needs_layout_passes=False),
        scratch_types=[
            pltpu.VMEM((csr_len,), jnp.int32),
            pltpu.VMEM((WIN * BLK,), jnp.int32),
            pltpu.VMEM((WIN * BLK,), jnp.int32),
            pltpu.VMEM((2, BLK), jnp.int32),
            pltpu.VMEM((BLK + 16,), jnp.int32),
            pltpu.VMEM((2, BLK, D), jnp.float32),
            pltpu.VMEM((BLK + 8, D), jnp.float32),
            pltpu.SemaphoreType.DMA,
            pltpu.SemaphoreType.DMA,
            pltpu.SemaphoreType.DMA,
            pltpu.SemaphoreType.DMA,
            pltpu.SemaphoreType.DMA,
        ],
    )
    def k(mod_hbm, fmi_hbm, csr_hbm, y_hbm, csr_v, idx_w0, idx_w1, idx_ov,
          seg_v, rows_v, out_buf, semg0, semg1, semi0, semi1, semo):
        wid = lax.axis_index("s") * nc + lax.axis_index("c")
        seg0 = wid * segw
        pltpu.sync_copy(csr_hbm.at[pl.ds(pl.multiple_of(seg0, 8), csr_len)],
                        csr_v)
        neg = jnp.full((16,), _NEG_INF, jnp.float32)
        zero = jnp.zeros((16,), jnp.float32)
        lanes = lax.iota(jnp.int32, 16)
        semg = (semg0, semg1)
        semi = (semi0, semi1)
        idx_ws = (idx_w0, idx_w1)

        def win_fetch(ea, par):
            pltpu.async_copy(
                fmi_hbm.at[pl.ds(pl.multiple_of(ea, 8), WIN * BLK)],
                idx_ws[par], semi[par])

        def win_wait(ea, par):
            pltpu.make_async_copy(
                fmi_hbm.at[pl.ds(pl.multiple_of(ea, 8), WIN * BLK)],
                idx_ws[par], semi[par]).wait()

        win_fetch((_sget(csr_v, 0) // 8) * 8, 0)

        def blockpair(bp, _):
            for b01 in range(2):
                b = 2 * bp + b01
                par = b01
                cb = b * BLK
                e0 = _sget(csr_v, cb)
                e1 = _sget(csr_v, cb + BLK)
                ea = (e0 // 8) * 8
                nch = (e1 - ea + (BLK - 1)) // BLK
                win_wait(ea, par)

                @pl.when(b + 1 < nblk)
                def _():
                    e0n = _sget(csr_v, cb + BLK)
                    win_fetch((e0n // 8) * 8, 1 - par)

                def issue(c, gbuf):
                    @pl.when(c < WIN)
                    def _():
                        pltpu.async_copy(
                            mod_hbm.at[idx_ws[par].at[pl.ds(c * BLK, BLK)]],
                            rows_v.at[gbuf], semg[gbuf])

                    @pl.when(c >= WIN)
                    def _():
                        pltpu.sync_copy(
                            fmi_hbm.at[pl.ds(
                                pl.multiple_of(ea + c * BLK, 8), BLK)],
                            idx_ov.at[gbuf])
                        pltpu.async_copy(mod_hbm.at[idx_ov.at[gbuf]],
                                         rows_v.at[gbuf], semg[gbuf])

                @pl.when(nch > 0)
                def _():
                    issue(0, 0)

                @pl.when(b > 0)
                def _():
                    pltpu.make_async_copy(
                        out_buf.at[pl.ds(0, BLK)],
                        y_hbm.at[pl.ds(seg0 + cb - BLK, BLK)], semo).wait()

                npair = (nch + 1) // 2

                def pairbody(g, carry):
                    for gbuf in range(2):
                        c = 2 * g + gbuf

                        @pl.when(c + 1 < nch)
                        def _():
                            issue(c + 1, 1 - gbuf)

                        @pl.when(c < nch)
                        def _():
                            pltpu.make_async_copy(
                                mod_hbm.at[idx_ov.at[gbuf]],
                                rows_v.at[gbuf], semg[gbuf]).wait()
                        base = ea + c * BLK
                        for g16 in range(BLK // 16):
                            iv = base + g16 * 16 + lanes
                            seg_v[pl.ds(g16 * 16, 16)] = _segids16(
                                csr_v, cb, iv, e0, e1)

                        def elem(j, st):
                            s_ = st[0]
                            acc = st[1:]
                            s_new = _sget(seg_v, j)
                            ch = s_new != s_
                            out = []
                            for kk in range(KSUB):
                                a = jnp.where(ch, neg, acc[kk])
                                a = jnp.maximum(
                                    a, rows_v[gbuf, j, pl.ds(kk * 16, 16)])
                                out_buf[s_new, pl.ds(kk * 16, 16)] = a
                                out.append(a)
                            return (s_new,) + tuple(out)

                        carry = lax.fori_loop(0, BLK, elem, carry, unroll=4)
                    return carry

                init = (jnp.int32(0),) + (neg,) * KSUB
                lax.fori_loop(0, npair, pairbody, init)

                def fix(r, _2):
                    @pl.when(_sget(csr_v, cb + r + 1) == _sget(csr_v, cb + r))
                    def _():
                        for kk in range(KSUB):
                            out_buf[r, pl.ds(kk * 16, 16)] = zero
                    return 0

                lax.fori_loop(0, BLK, fix, 0)
                pltpu.async_copy(out_buf.at[pl.ds(0, BLK)],
                                 y_hbm.at[pl.ds(seg0 + cb, BLK)], semo)
            return 0

        lax.fori_loop(0, nblk // 2, blockpair, 0)
        pltpu.make_async_copy(
            out_buf.at[pl.ds(0, BLK)],
            y_hbm.at[pl.ds(seg0 + (nblk - 1) * BLK, BLK)], semo).wait()

    return k(mod_x, fmi_pad, csr_pad)


def _view_pool_sc(y, scores_pad, vcsr, n_pts):
    info = plsc.get_sparse_core_info()
    nc, ns = info.num_cores, info.num_subcores
    nw = nc * ns
    ptw = n_pts // nw
    nblk = ptw // BLK
    csr_len = ptw + 24

    csr_pad = jnp.pad(vcsr, (0, 32), mode="edge")
    WIN = 8

    @functools.partial(
        pl.kernel,
        mesh=plsc.VectorSubcoreMesh(core_axis_name="c", subcore_axis_name="s"),
        out_type=jax.ShapeDtypeStruct((n_pts, D), jnp.float32),
        compiler_params=pltpu.CompilerParams(needs_layout_passes=False),
        scratch_types=[
            pltpu.VMEM((csr_len,), jnp.int32),
            pltpu.VMEM((WIN * BLK,), jnp.float32),
            pltpu.VMEM((WIN * BLK,), jnp.float32),
            pltpu.VMEM((BLK + 16,), jnp.int32),
            pltpu.VMEM((2, BLK, D), jnp.float32),
            pltpu.VMEM((BLK + 8, D), jnp.float32),
            pltpu.SemaphoreType.DMA,
            pltpu.SemaphoreType.DMA,
            pltpu.SemaphoreType.DMA,
            pltpu.SemaphoreType.DMA,
            pltpu.SemaphoreType.DMA,
        ],
    )
    def k(y_hbm, sc_hbm, csr_hbm, out_hbm, csr_v, s_w0, s_w1, seg_v, rows_v,
          out_buf, semg0, semg1, semi0, semi1, semo):
        wid = lax.axis_index("s") * nc + lax.axis_index("c")
        pt0 = wid * ptw
        pltpu.sync_copy(csr_hbm.at[pl.ds(pl.multiple_of(pt0, 8), csr_len)],
                        csr_v)
        negv = jnp.full((16,), _NEG_INF, jnp.float32)
        zero = jnp.zeros((16,), jnp.float32)
        eps = jnp.full((16,), 1e-12, jnp.float32)
        lanes = lax.iota(jnp.int32, 16)
        semg = (semg0, semg1)
        semi = (semi0, semi1)
        s_ws = (s_w0, s_w1)

        def win_fetch(va, par):
            pltpu.async_copy(
                sc_hbm.at[pl.ds(pl.multiple_of(va, 8), WIN * BLK)],
                s_ws[par], semi[par])

        def win_wait(va, par):
            pltpu.make_async_copy(
                sc_hbm.at[pl.ds(pl.multiple_of(va, 8), WIN * BLK)],
                s_ws[par], semi[par]).wait()

        win_fetch((_sget(csr_v, 0) // 8) * 8, 0)

        def blockpair(bp, _):
            for b01 in range(2):
                b = 2 * bp + b01
                par = b01
                cb = b * BLK
                v0 = _sget(csr_v, cb)
                v1 = _sget(csr_v, cb + BLK)
                ea = (v0 // 8) * 8
                nch = (v1 - ea + (BLK - 1)) // BLK
                win_wait(ea, par)

                @pl.when(b + 1 < nblk)
                def _():
                    v0n = _sget(csr_v, cb + BLK)
                    win_fetch((v0n // 8) * 8, 1 - par)

                def issue(c, gbuf):
                    @pl.when(c >= WIN)
                    def _():
                        pltpu.sync_copy(
                            sc_hbm.at[pl.ds(
                                pl.multiple_of(ea + c * BLK, 8), BLK)],
                            s_ws[par].at[pl.ds((c % WIN) * BLK, BLK)])
                    pltpu.async_copy(y_hbm.at[pl.ds(ea + c * BLK, BLK)],
                                     rows_v.at[gbuf], semg[gbuf])

                @pl.when(nch > 0)
                def _():
                    issue(0, 0)

                @pl.when(b > 0)
                def _():
                    pltpu.make_async_copy(
                        out_buf.at[pl.ds(0, BLK)],
                        out_hbm.at[pl.ds(pt0 + cb - BLK, BLK)], semo).wait()

                npair = (nch + 1) // 2

                def pairbody(g, carry):
                    for gbuf in range(2):
                        c = 2 * g + gbuf

                        @pl.when(c + 1 < nch)
                        def _():
                            issue(c + 1, 1 - gbuf)

                        @pl.when(c < nch)
                        def _():
                            pltpu.make_async_copy(
                                y_hbm.at[pl.ds(ea, BLK)], rows_v.at[gbuf],
                                semg[gbuf]).wait()
                        base = ea + c * BLK
                        woff = (c % WIN) * BLK
                        for g16 in range(BLK // 16):
                            iv = base + g16 * 16 + lanes
                            seg_v[pl.ds(g16 * 16, 16)] = _segids16(
                                csr_v, cb, iv, v0, v1)

                        def elem(j, st):
                            p_ = st[0]
                            m_, d_ = st[1], st[2]
                            acc = st[3:]
                            p_new = _sget(seg_v, j)
                            ch = p_new != p_
                            m2 = jnp.where(ch, negv, m_)
                            d2 = jnp.where(ch, zero, d_)
                            s_b = plsc.load_gather(
                                s_ws[par],
                                [jnp.full((16,), woff + j, jnp.int32)])
                            m_new = jnp.maximum(m2, s_b)
                            scale = jnp.exp(m2 - m_new)
                            e = jnp.exp(s_b - m_new)
                            d_new = d2 * scale + e
                            inv = 1.0 / (d_new + eps)
                            out = []
                            for kk in range(KSUB):
                                a = jnp.where(ch, zero, acc[kk])
                                a = a * scale + e * rows_v[gbuf, j,
                                                           pl.ds(kk * 16, 16)]
                                out_buf[p_new, pl.ds(kk * 16, 16)] = a * inv
                                out.append(a)
                            return (p_new, m_new, d_new) + tuple(out)

                        carry = lax.fori_loop(0, BLK, elem, carry, unroll=4)
                    return carry

                init = (jnp.int32(0), negv, zero) + (zero,) * KSUB
                lax.fori_loop(0, npair, pairbody, init)

                def fix(r, _2):
                    @pl.when(_sget(csr_v, cb + r + 1) == _sget(csr_v, cb + r))
                    def _():
                        for kk in range(KSUB):
                            out_buf[r, pl.ds(kk * 16, 16)] = zero
                    return 0

                lax.fori_loop(0, BLK, fix, 0)
                pltpu.async_copy(out_buf.at[pl.ds(0, BLK)],
                                 out_hbm.at[pl.ds(pt0 + cb, BLK)], semo)
            return 0

        lax.fori_loop(0, nblk // 2, blockpair, 0)
        pltpu.make_async_copy(
            out_buf.at[pl.ds(0, BLK)],
            out_hbm.at[pl.ds(pt0 + (nblk - 1) * BLK, BLK)], semo).wait()

    return k(y, scores_pad, csr_pad)


def kernel(x_3d, mod_x, feature_map_indexing, atomic_csr, view_csr, x_map,
           W_fuse, b_fuse, w_att):
    n_view = x_map.shape[0]
    n_pts = x_3d.shape[0]

    scores = _scores_tc(x_map, w_att)[:, 0]
    scores_pad = jnp.pad(scores, (0, 2048))

    y = _atomic_pool_sc(mod_x, feature_map_indexing, atomic_csr, n_view)
    x_pool = _view_pool_sc(y, scores_pad, view_csr, n_pts)

    fused = _fuse_tc(x_3d, x_pool, W_fuse, b_fuse)
    x_seen = view_csr[1:] > view_csr[:-1]
    return fused, x_seen


# speedup vs baseline: 2.3551x; 2.3551x over previous
import functools

import jax
import jax.numpy as jnp
from jax import lax
from jax.experimental import pallas as pl
from jax.experimental.pallas import tpu as pltpu
from jax.experimental.pallas import tpu_sc as plsc

D = 256
KSUB = D // 16
BLK = 128

_NEG_INF = float("-inf")


def _sget(ref, idx):
    return ref[pl.ds(idx, 16)][0]


_BSTEPS = (64, 32, 16, 8, 4, 2, 1)


def _segids16(csr_v, cb, iv, e0, e1):
    lo = jnp.zeros((16,), jnp.int32)
    for step in _BSTEPS:
        cand = lo + step
        v = plsc.load_gather(csr_v, [cb + cand])
        lo = jnp.where(v <= iv, cand, lo)
    return jnp.where((iv >= e0) & (iv < e1), lo, BLK)


def _scores_tc(x_map, w_att):
    n, d = x_map.shape
    blk = 1024

    def body(xm, w, o):
        o[...] = lax.dot_general(
            xm[...], w[...], (((1,), (0,)), ((), ())),
            preferred_element_type=jnp.float32)

    return pl.pallas_call(
        body,
        grid=(n // blk,),
        in_specs=[
            pl.BlockSpec((blk, d), lambda i: (i, 0)),
            pl.BlockSpec((d, 1), lambda i: (0, 0)),
        ],
        out_specs=pl.BlockSpec((blk, 1), lambda i: (i, 0)),
        out_shape=jax.ShapeDtypeStruct((n, 1), jnp.float32),
    )(x_map, w_att.reshape(d, 1))


def _fuse_tc(x_3d, x_pool, W_fuse, b_fuse):
    n, d = x_3d.shape
    blk = 1024
    w1 = W_fuse[:d]
    w2 = W_fuse[d:]
    b2 = b_fuse.reshape(1, d)

    def body(a, p, wa, wb, b, o):
        acc = lax.dot_general(a[...], wa[...], (((1,), (0,)), ((), ())),
                              preferred_element_type=jnp.float32)
        acc += lax.dot_general(p[...], wb[...], (((1,), (0,)), ((), ())),
                               preferred_element_type=jnp.float32)
        o[...] = acc + b[...]

    return pl.pallas_call(
        body,
        grid=(n // blk,),
        in_specs=[
            pl.BlockSpec((blk, d), lambda i: (i, 0)),
            pl.BlockSpec((blk, d), lambda i: (i, 0)),
            pl.BlockSpec((d, d), lambda i: (0, 0)),
            pl.BlockSpec((d, d), lambda i: (0, 0)),
            pl.BlockSpec((1, d), lambda i: (0, 0)),
        ],
        out_specs=pl.BlockSpec((blk, d), lambda i: (i, 0)),
        out_shape=jax.ShapeDtypeStruct((n, d), jnp.float32),
    )(x_3d, x_pool, w1, w2, b2)


def _atomic_pool_sc(mod_x, fmi, acsr, n_seg):
    info = plsc.get_sparse_core_info()
    nc, ns = info.num_cores, info.num_subcores
    nw = nc * ns
    segw = n_seg // nw
    nblk = segw // BLK
    csr_len = segw + 24
    WIN = 8

    fmi_pad = jnp.pad(fmi, (0, (WIN + 1) * BLK))
    csr_pad = jnp.pad(acsr, (0, 32), mode="edge")

    @functools.partial(
        pl.kernel,
        mesh=plsc.VectorSubcoreMesh(core_axis_name="c", subcore_axis_name="s"),
        out_type=jax.ShapeDtypeStruct((n_seg + 128, D), jnp.float32),
        compiler_params=pltpu.CompilerParams(needs_layout_passes=False),
        scratch_types=[
            pltpu.VMEM((csr_len,), jnp.int32),
            pltpu.VMEM((WIN * BLK,), jnp.int32),
            pltpu.VMEM((WIN * BLK,), jnp.int32),
            pltpu.VMEM((2, BLK), jnp.int32),
            pltpu.VMEM((BLK + 16,), jnp.int32),
            pltpu.VMEM((2, BLK, D), jnp.float32),
            pltpu.VMEM((BLK + 8, D), jnp.float32),
            pltpu.SemaphoreType.DMA,
            pltpu.SemaphoreType.DMA,
            pltpu.SemaphoreType.DMA,
            pltpu.SemaphoreType.DMA,
            pltpu.SemaphoreType.DMA,
        ],
    )
    def k(mod_hbm, fmi_hbm, csr_hbm, y_hbm, csr_v, idx_w0, idx_w1, idx_ov,
          seg_v, rows_v, out_buf, semg0, semg1, semi0, semi1, semo):
        wid = lax.axis_index("s") * nc + lax.axis_index("c")
        seg0 = wid * segw
        pltpu.sync_copy(csr_hbm.at[pl.ds(pl.multiple_of(seg0, 8), csr_len)],
                        csr_v)
        neg = jnp.full((16,), _NEG_INF, jnp.float32)
        zero = jnp.zeros((16,), jnp.float32)
        lanes = lax.iota(jnp.int32, 16)
        semg = (semg0, semg1)
        semi = (semi0, semi1)
        idx_ws = (idx_w0, idx_w1)

        def win_fetch(ea, par):
            pltpu.async_copy(
                fmi_hbm.at[pl.ds(pl.multiple_of(ea, 8), WIN * BLK)],
                idx_ws[par], semi[par])

        def win_wait(ea, par):
            pltpu.make_async_copy(
                fmi_hbm.at[pl.ds(pl.multiple_of(ea, 8), WIN * BLK)],
                idx_ws[par], semi[par]).wait()

        win_fetch((_sget(csr_v, 0) // 8) * 8, 0)

        def blockpair(bp, _):
            for b01 in range(2):
                b = 2 * bp + b01
                par = b01
                cb = b * BLK
                e0 = _sget(csr_v, cb)
                e1 = _sget(csr_v, cb + BLK)
                ea = (e0 // 8) * 8
                nch = (e1 - ea + (BLK - 1)) // BLK
                win_wait(ea, par)

                @pl.when(b + 1 < nblk)
                def _():
                    e0n = _sget(csr_v, cb + BLK)
                    win_fetch((e0n // 8) * 8, 1 - par)

                def issue(c, gbuf):
                    @pl.when(c < WIN)
                    def _():
                        pltpu.async_copy(
                            mod_hbm.at[idx_ws[par].at[pl.ds(c * BLK, BLK)]],
                            rows_v.at[gbuf], semg[gbuf])

                    @pl.when(c >= WIN)
                    def _():
                        pltpu.sync_copy(
                            fmi_hbm.at[pl.ds(
                                pl.multiple_of(ea + c * BLK, 8), BLK)],
                            idx_ov.at[gbuf])
                        pltpu.async_copy(mod_hbm.at[idx_ov.at[gbuf]],
                                         rows_v.at[gbuf], semg[gbuf])

                @pl.when(nch > 0)
                def _():
                    issue(0, 0)

                @pl.when(b > 0)
                def _():
                    pltpu.make_async_copy(
                        out_buf.at[pl.ds(0, BLK)],
                        y_hbm.at[pl.ds(seg0 + cb - BLK, BLK)], semo).wait()

                npair = (nch + 1) // 2

                def pairbody(g, carry):
                    for gbuf in range(2):
                        c = 2 * g + gbuf

                        @pl.when(c + 1 < nch)
                        def _():
                            issue(c + 1, 1 - gbuf)

                        @pl.when(c < nch)
                        def _():
                            pltpu.make_async_copy(
                                mod_hbm.at[idx_ov.at[gbuf]],
                                rows_v.at[gbuf], semg[gbuf]).wait()
                        base = ea + c * BLK
                        for g16 in range(BLK // 16):
                            iv = base + g16 * 16 + lanes
                            seg_v[pl.ds(g16 * 16, 16)] = _segids16(
                                csr_v, cb, iv, e0, e1)

                        j0 = jnp.maximum(e0, base) - base
                        j1 = jnp.maximum(
                            j0, jnp.minimum(e1, base + BLK) - base)

                        def elem(j, st):
                            s_ = st[0]
                            acc = st[1:]
                            s_new = _sget(seg_v, j)
                            ch = s_new != s_

                            @pl.when(ch)
                            def _():
                                for kk in range(KSUB):
                                    out_buf[s_, pl.ds(kk * 16, 16)] = acc[kk]

                            out = []
                            for kk in range(KSUB):
                                a = jnp.where(ch, neg, acc[kk])
                                a = jnp.maximum(
                                    a, rows_v[gbuf, j, pl.ds(kk * 16, 16)])
                                out.append(a)
                            return (s_new,) + tuple(out)

                        carry = lax.fori_loop(j0, j1, elem, carry)
                    return carry

                init = (jnp.int32(0),) + (neg,) * KSUB
                st = lax.fori_loop(0, npair, pairbody, init)
                s_fin = st[0]
                acc_fin = st[1:]
                for kk in range(KSUB):
                    out_buf[s_fin, pl.ds(kk * 16, 16)] = acc_fin[kk]

                def fix(r, _2):
                    @pl.when(_sget(csr_v, cb + r + 1) == _sget(csr_v, cb + r))
                    def _():
                        for kk in range(KSUB):
                            out_buf[r, pl.ds(kk * 16, 16)] = zero
                    return 0

                lax.fori_loop(0, BLK, fix, 0)
                pltpu.async_copy(out_buf.at[pl.ds(0, BLK)],
                                 y_hbm.at[pl.ds(seg0 + cb, BLK)], semo)
            return 0

        lax.fori_loop(0, nblk // 2, blockpair, 0)
        pltpu.make_async_copy(
            out_buf.at[pl.ds(0, BLK)],
            y_hbm.at[pl.ds(seg0 + (nblk - 1) * BLK, BLK)], semo).wait()

    return k(mod_x, fmi_pad, csr_pad)


def _view_pool_sc(y, scores_pad, vcsr, n_pts):
    info = plsc.get_sparse_core_info()
    nc, ns = info.num_cores, info.num_subcores
    nw = nc * ns
    ptw = n_pts // nw
    nblk = ptw // BLK
    csr_len = ptw + 24

    csr_pad = jnp.pad(vcsr, (0, 32), mode="edge")
    WIN = 8

    @functools.partial(
        pl.kernel,
        mesh=plsc.VectorSubcoreMesh(core_axis_name="c", subcore_axis_name="s"),
        out_type=jax.ShapeDtypeStruct((n_pts, D), jnp.float32),
        compiler_params=pltpu.CompilerParams(needs_layout_passes=False),
        scratch_types=[
            pltpu.VMEM((csr_len,), jnp.int32),
            pltpu.VMEM((WIN * BLK,), jnp.float32),
            pltpu.VMEM((WIN * BLK,), jnp.float32),
            pltpu.VMEM((BLK + 16,), jnp.int32),
            pltpu.VMEM((2, BLK, D), jnp.float32),
            pltpu.VMEM((BLK + 8, D), jnp.float32),
            pltpu.SemaphoreType.DMA,
            pltpu.SemaphoreType.DMA,
            pltpu.SemaphoreType.DMA,
            pltpu.SemaphoreType.DMA,
            pltpu.SemaphoreType.DMA,
        ],
    )
    def k(y_hbm, sc_hbm, csr_hbm, out_hbm, csr_v, s_w0, s_w1, seg_v, rows_v,
          out_buf, semg0, semg1, semi0, semi1, semo):
        wid = lax.axis_index("s") * nc + lax.axis_index("c")
        pt0 = wid * ptw
        pltpu.sync_copy(csr_hbm.at[pl.ds(pl.multiple_of(pt0, 8), csr_len)],
                        csr_v)
        negv = jnp.full((16,), _NEG_INF, jnp.float32)
        zero = jnp.zeros((16,), jnp.float32)
        eps = jnp.full((16,), 1e-12, jnp.float32)
        lanes = lax.iota(jnp.int32, 16)
        semg = (semg0, semg1)
        semi = (semi0, semi1)
        s_ws = (s_w0, s_w1)

        def win_fetch(va, par):
            pltpu.async_copy(
                sc_hbm.at[pl.ds(pl.multiple_of(va, 8), WIN * BLK)],
                s_ws[par], semi[par])

        def win_wait(va, par):
            pltpu.make_async_copy(
                sc_hbm.at[pl.ds(pl.multiple_of(va, 8), WIN * BLK)],
                s_ws[par], semi[par]).wait()

        win_fetch((_sget(csr_v, 0) // 8) * 8, 0)

        def blockpair(bp, _):
            for b01 in range(2):
                b = 2 * bp + b01
                par = b01
                cb = b * BLK
                v0 = _sget(csr_v, cb)
                v1 = _sget(csr_v, cb + BLK)
                ea = (v0 // 8) * 8
                nch = (v1 - ea + (BLK - 1)) // BLK
                win_wait(ea, par)

                @pl.when(b + 1 < nblk)
                def _():
                    v0n = _sget(csr_v, cb + BLK)
                    win_fetch((v0n // 8) * 8, 1 - par)

                def issue(c, gbuf):
                    @pl.when(c >= WIN)
                    def _():
                        pltpu.sync_copy(
                            sc_hbm.at[pl.ds(
                                pl.multiple_of(ea + c * BLK, 8), BLK)],
                            s_ws[par].at[pl.ds((c % WIN) * BLK, BLK)])
                    pltpu.async_copy(y_hbm.at[pl.ds(ea + c * BLK, BLK)],
                                     rows_v.at[gbuf], semg[gbuf])

                @pl.when(nch > 0)
                def _():
                    issue(0, 0)

                @pl.when(b > 0)
                def _():
                    pltpu.make_async_copy(
                        out_buf.at[pl.ds(0, BLK)],
                        out_hbm.at[pl.ds(pt0 + cb - BLK, BLK)], semo).wait()

                npair = (nch + 1) // 2

                def pairbody(g, carry):
                    for gbuf in range(2):
                        c = 2 * g + gbuf

                        @pl.when(c + 1 < nch)
                        def _():
                            issue(c + 1, 1 - gbuf)

                        @pl.when(c < nch)
                        def _():
                            pltpu.make_async_copy(
                                y_hbm.at[pl.ds(ea, BLK)], rows_v.at[gbuf],
                                semg[gbuf]).wait()
                        base = ea + c * BLK
                        woff = (c % WIN) * BLK
                        for g16 in range(BLK // 16):
                            iv = base + g16 * 16 + lanes
                            seg_v[pl.ds(g16 * 16, 16)] = _segids16(
                                csr_v, cb, iv, v0, v1)

                        j0 = jnp.maximum(v0, base) - base
                        j1 = jnp.maximum(
                            j0, jnp.minimum(v1, base + BLK) - base)

                        def elem(j, st):
                            p_ = st[0]
                            m_, d_ = st[1], st[2]
                            acc = st[3:]
                            p_new = _sget(seg_v, j)
                            ch = p_new != p_

                            @pl.when(ch)
                            def _():
                                inv = 1.0 / (d_ + eps)
                                for kk in range(KSUB):
                                    out_buf[p_, pl.ds(kk * 16, 16)] = \
                                        acc[kk] * inv

                            m2 = jnp.where(ch, negv, m_)
                            d2 = jnp.where(ch, zero, d_)
                            s_b = plsc.load_gather(
                                s_ws[par],
                                [jnp.full((16,), woff + j, jnp.int32)])
                            m_new = jnp.maximum(m2, s_b)
                            scale = jnp.exp(m2 - m_new)
                            e = jnp.exp(s_b - m_new)
                            d_new = d2 * scale + e
                            out = []
                            for kk in range(KSUB):
                                a = jnp.where(ch, zero, acc[kk])
                                a = a * scale + e * rows_v[gbuf, j,
                                                           pl.ds(kk * 16, 16)]
                                out.append(a)
                            return (p_new, m_new, d_new) + tuple(out)

                        carry = lax.fori_loop(j0, j1, elem, carry)
                    return carry

                init = (jnp.int32(0), negv, zero) + (zero,) * KSUB
                st = lax.fori_loop(0, npair, pairbody, init)
                p_fin = st[0]
                d_fin = st[2]
                acc_fin = st[3:]
                inv_fin = 1.0 / (d_fin + eps)
                for kk in range(KSUB):
                    out_buf[p_fin, pl.ds(kk * 16, 16)] = acc_fin[kk] * inv_fin

                def fix(r, _2):
                    @pl.when(_sget(csr_v, cb + r + 1) == _sget(csr_v, cb + r))
                    def _():
                        for kk in range(KSUB):
                            out_buf[r, pl.ds(kk * 16, 16)] = zero
                    return 0

                lax.fori_loop(0, BLK, fix, 0)
                pltpu.async_copy(out_buf.at[pl.ds(0, BLK)],
                                 out_hbm.at[pl.ds(pt0 + cb, BLK)], semo)
            return 0

        lax.fori_loop(0, nblk // 2, blockpair, 0)
        pltpu.make_async_copy(
            out_buf.at[pl.ds(0, BLK)],
            out_hbm.at[pl.ds(pt0 + (nblk - 1) * BLK, BLK)], semo).wait()

    return k(y, scores_pad, csr_pad)


def kernel(x_3d, mod_x, feature_map_indexing, atomic_csr, view_csr, x_map,
           W_fuse, b_fuse, w_att):
    n_view = x_map.shape[0]
    n_pts = x_3d.shape[0]

    scores = _scores_tc(x_map, w_att)[:, 0]
    scores_pad = jnp.pad(scores, (0, 2048))

    y = _atomic_pool_sc(mod_x, feature_map_indexing, atomic_csr, n_view)
    x_pool = _view_pool_sc(y, scores_pad, view_csr, n_pts)

    fused = _fuse_tc(x_3d, x_pool, W_fuse, b_fuse)
    x_seen = view_csr[1:] > view_csr[:-1]
    return fused, x_seen


# speedup vs baseline: 2.6156x; 1.1106x over previous
import functools

import jax
import jax.numpy as jnp
from jax import lax
from jax.experimental import pallas as pl
from jax.experimental.pallas import tpu as pltpu
from jax.experimental.pallas import tpu_sc as plsc

D = 256
KSUB = D // 16
BLK = 128

_NEG_INF = float("-inf")


def _sget(ref, idx):
    return ref[pl.ds(idx, 16)][0]


_BSTEPS = (64, 32, 16, 8, 4, 2, 1)


def _segids16(csr_v, cb, iv, e0, e1):
    lo = jnp.zeros((16,), jnp.int32)
    for step in _BSTEPS:
        cand = lo + step
        v = plsc.load_gather(csr_v, [cb + cand])
        lo = jnp.where(v <= iv, cand, lo)
    return jnp.where((iv >= e0) & (iv < e1), lo, BLK)


def _scores_tc(x_map, w_att):
    n, d = x_map.shape
    blk = 1024

    def body(xm, w, o):
        o[...] = lax.dot_general(
            xm[...], w[...], (((1,), (0,)), ((), ())),
            preferred_element_type=jnp.float32)

    return pl.pallas_call(
        body,
        grid=(n // blk,),
        in_specs=[
            pl.BlockSpec((blk, d), lambda i: (i, 0)),
            pl.BlockSpec((d, 1), lambda i: (0, 0)),
        ],
        out_specs=pl.BlockSpec((blk, 1), lambda i: (i, 0)),
        out_shape=jax.ShapeDtypeStruct((n, 1), jnp.float32),
    )(x_map, w_att.reshape(d, 1))


def _fuse_tc(x_3d, x_pool, W_fuse, b_fuse):
    n, d = x_3d.shape
    blk = 1024
    w1 = W_fuse[:d]
    w2 = W_fuse[d:]
    b2 = b_fuse.reshape(1, d)

    def body(a, p, wa, wb, b, o):
        acc = lax.dot_general(a[...], wa[...], (((1,), (0,)), ((), ())),
                              preferred_element_type=jnp.float32)
        acc += lax.dot_general(p[...], wb[...], (((1,), (0,)), ((), ())),
                               preferred_element_type=jnp.float32)
        o[...] = acc + b[...]

    return pl.pallas_call(
        body,
        grid=(n // blk,),
        in_specs=[
            pl.BlockSpec((blk, d), lambda i: (i, 0)),
            pl.BlockSpec((blk, d), lambda i: (i, 0)),
            pl.BlockSpec((d, d), lambda i: (0, 0)),
            pl.BlockSpec((d, d), lambda i: (0, 0)),
            pl.BlockSpec((1, d), lambda i: (0, 0)),
        ],
        out_specs=pl.BlockSpec((blk, d), lambda i: (i, 0)),
        out_shape=jax.ShapeDtypeStruct((n, d), jnp.float32),
    )(x_3d, x_pool, w1, w2, b2)


def _atomic_pool_sc(mod_x, fmi, acsr, n_seg):
    info = plsc.get_sparse_core_info()
    nc, ns = info.num_cores, info.num_subcores
    nw = nc * ns
    segw = n_seg // nw
    nblk = segw // BLK
    csr_len = segw + 24
    WIN = 8

    fmi_pad = jnp.pad(fmi, (0, (WIN + 1) * BLK))
    csr_pad = jnp.pad(acsr, (0, 32), mode="edge")

    @functools.partial(
        pl.kernel,
        mesh=plsc.VectorSubcoreMesh(core_axis_name="c", subcore_axis_name="s"),
        out_type=jax.ShapeDtypeStruct((n_seg + 128, D), jnp.float32),
        compiler_params=pltpu.CompilerParams(needs_layout_passes=False),
        scratch_types=[
            pltpu.VMEM((csr_len,), jnp.int32),
            pltpu.VMEM((WIN * BLK,), jnp.int32),
            pltpu.VMEM((WIN * BLK,), jnp.int32),
            pltpu.VMEM((2, BLK), jnp.int32),
            pltpu.VMEM((BLK + 16,), jnp.int32),
            pltpu.VMEM((2, BLK, D), jnp.float32),
            pltpu.VMEM((BLK + 8, D), jnp.float32),
            pltpu.SemaphoreType.DMA,
            pltpu.SemaphoreType.DMA,
            pltpu.SemaphoreType.DMA,
            pltpu.SemaphoreType.DMA,
            pltpu.SemaphoreType.DMA,
        ],
    )
    def k(mod_hbm, fmi_hbm, csr_hbm, y_hbm, csr_v, idx_w0, idx_w1, idx_ov,
          seg_v, rows_v, out_buf, semg0, semg1, semi0, semi1, semo):
        wid = lax.axis_index("s") * nc + lax.axis_index("c")
        seg0 = wid * segw
        pltpu.sync_copy(csr_hbm.at[pl.ds(pl.multiple_of(seg0, 8), csr_len)],
                        csr_v)
        neg = jnp.full((16,), _NEG_INF, jnp.float32)
        zero = jnp.zeros((16,), jnp.float32)
        lanes = lax.iota(jnp.int32, 16)
        semg = (semg0, semg1)
        semi = (semi0, semi1)
        idx_ws = (idx_w0, idx_w1)

        def win_fetch(ea, par):
            pltpu.async_copy(
                fmi_hbm.at[pl.ds(pl.multiple_of(ea, 8), WIN * BLK)],
                idx_ws[par], semi[par])

        def win_wait(ea, par):
            pltpu.make_async_copy(
                fmi_hbm.at[pl.ds(pl.multiple_of(ea, 8), WIN * BLK)],
                idx_ws[par], semi[par]).wait()

        win_fetch((_sget(csr_v, 0) // 8) * 8, 0)

        def blockpair(bp, _):
            for b01 in range(2):
                b = 2 * bp + b01
                par = b01
                cb = b * BLK
                e0 = _sget(csr_v, cb)
                e1 = _sget(csr_v, cb + BLK)
                ea = (e0 // 8) * 8
                nch = (e1 - ea + (BLK - 1)) // BLK
                win_wait(ea, par)

                @pl.when(b + 1 < nblk)
                def _():
                    e0n = _sget(csr_v, cb + BLK)
                    win_fetch((e0n // 8) * 8, 1 - par)

                def issue(c, gbuf):
                    @pl.when(c < WIN)
                    def _():
                        pltpu.async_copy(
                            mod_hbm.at[idx_ws[par].at[pl.ds(c * BLK, BLK)]],
                            rows_v.at[gbuf], semg[gbuf])

                    @pl.when(c >= WIN)
                    def _():
                        pltpu.sync_copy(
                            fmi_hbm.at[pl.ds(
                                pl.multiple_of(ea + c * BLK, 8), BLK)],
                            idx_ov.at[gbuf])
                        pltpu.async_copy(mod_hbm.at[idx_ov.at[gbuf]],
                                         rows_v.at[gbuf], semg[gbuf])

                @pl.when(nch > 0)
                def _():
                    issue(0, 0)

                @pl.when(b > 0)
                def _():
                    pltpu.make_async_copy(
                        out_buf.at[pl.ds(0, BLK)],
                        y_hbm.at[pl.ds(seg0 + cb - BLK, BLK)], semo).wait()

                npair = (nch + 1) // 2

                def pairbody(g, carry):
                    for gbuf in range(2):
                        c = 2 * g + gbuf

                        @pl.when(c + 1 < nch)
                        def _():
                            issue(c + 1, 1 - gbuf)

                        @pl.when(c < nch)
                        def _():
                            pltpu.make_async_copy(
                                mod_hbm.at[idx_ov.at[gbuf]],
                                rows_v.at[gbuf], semg[gbuf]).wait()
                        base = ea + c * BLK
                        for g16 in range(BLK // 16):
                            iv = base + g16 * 16 + lanes
                            seg_v[pl.ds(g16 * 16, 16)] = _segids16(
                                csr_v, cb, iv, e0, e1)

                        def group(gi, st):
                            segv = seg_v[pl.ds(gi * 16, 16)]
                            for u in range(16):
                                s_ = st[0]
                                acc = st[1:]
                                s_new = segv[u]
                                ch = s_new != s_

                                @pl.when(ch)
                                def _():
                                    for kk in range(KSUB):
                                        out_buf[s_, pl.ds(kk * 16, 16)] = \
                                            acc[kk]

                                out = []
                                for kk in range(KSUB):
                                    a = jnp.where(ch, neg, acc[kk])
                                    a = jnp.maximum(
                                        a, rows_v[gbuf, gi * 16 + u,
                                                  pl.ds(kk * 16, 16)])
                                    out.append(a)
                                st = (s_new,) + tuple(out)
                            return st

                        carry = lax.fori_loop(0, BLK // 16, group, carry)
                    return carry

                init = (jnp.int32(0),) + (neg,) * KSUB
                st = lax.fori_loop(0, npair, pairbody, init)
                s_fin = st[0]
                acc_fin = st[1:]
                for kk in range(KSUB):
                    out_buf[s_fin, pl.ds(kk * 16, 16)] = acc_fin[kk]

                def fix(r, _2):
                    @pl.when(_sget(csr_v, cb + r + 1) == _sget(csr_v, cb + r))
                    def _():
                        for kk in range(KSUB):
                            out_buf[r, pl.ds(kk * 16, 16)] = zero
                    return 0

                lax.fori_loop(0, BLK, fix, 0)
                pltpu.async_copy(out_buf.at[pl.ds(0, BLK)],
                                 y_hbm.at[pl.ds(seg0 + cb, BLK)], semo)
            return 0

        lax.fori_loop(0, nblk // 2, blockpair, 0)
        pltpu.make_async_copy(
            out_buf.at[pl.ds(0, BLK)],
            y_hbm.at[pl.ds(seg0 + (nblk - 1) * BLK, BLK)], semo).wait()

    return k(mod_x, fmi_pad, csr_pad)


def _view_pool_sc(y, scores_pad, vcsr, n_pts):
    info = plsc.get_sparse_core_info()
    nc, ns = info.num_cores, info.num_subcores
    nw = nc * ns
    ptw = n_pts // nw
    nblk = ptw // BLK
    csr_len = ptw + 24

    csr_pad = jnp.pad(vcsr, (0, 32), mode="edge")
    WIN = 8

    @functools.partial(
        pl.kernel,
        mesh=plsc.VectorSubcoreMesh(core_axis_name="c", subcore_axis_name="s"),
        out_type=jax.ShapeDtypeStruct((n_pts, D), jnp.float32),
        compiler_params=pltpu.CompilerParams(needs_layout_passes=False),
        scratch_types=[
            pltpu.VMEM((csr_len,), jnp.int32),
            pltpu.VMEM((WIN * BLK,), jnp.float32),
            pltpu.VMEM((WIN * BLK,), jnp.float32),
            pltpu.VMEM((BLK + 16,), jnp.int32),
            pltpu.VMEM((2, BLK, D), jnp.float32),
            pltpu.VMEM((BLK + 8, D), jnp.float32),
            pltpu.SemaphoreType.DMA,
            pltpu.SemaphoreType.DMA,
            pltpu.SemaphoreType.DMA,
            pltpu.SemaphoreType.DMA,
            pltpu.SemaphoreType.DMA,
        ],
    )
    def k(y_hbm, sc_hbm, csr_hbm, out_hbm, csr_v, s_w0, s_w1, seg_v, rows_v,
          out_buf, semg0, semg1, semi0, semi1, semo):
        wid = lax.axis_index("s") * nc + lax.axis_index("c")
        pt0 = wid * ptw
        pltpu.sync_copy(csr_hbm.at[pl.ds(pl.multiple_of(pt0, 8), csr_len)],
                        csr_v)
        negv = jnp.full((16,), _NEG_INF, jnp.float32)
        zero = jnp.zeros((16,), jnp.float32)
        eps = jnp.full((16,), 1e-12, jnp.float32)
        lanes = lax.iota(jnp.int32, 16)
        semg = (semg0, semg1)
        semi = (semi0, semi1)
        s_ws = (s_w0, s_w1)

        def win_fetch(va, par):
            pltpu.async_copy(
                sc_hbm.at[pl.ds(pl.multiple_of(va, 8), WIN * BLK)],
                s_ws[par], semi[par])

        def win_wait(va, par):
            pltpu.make_async_copy(
                sc_hbm.at[pl.ds(pl.multiple_of(va, 8), WIN * BLK)],
                s_ws[par], semi[par]).wait()

        win_fetch((_sget(csr_v, 0) // 8) * 8, 0)

        def blockpair(bp, _):
            for b01 in range(2):
                b = 2 * bp + b01
                par = b01
                cb = b * BLK
                v0 = _sget(csr_v, cb)
                v1 = _sget(csr_v, cb + BLK)
                ea = (v0 // 8) * 8
                nch = (v1 - ea + (BLK - 1)) // BLK
                win_wait(ea, par)

                @pl.when(b + 1 < nblk)
                def _():
                    v0n = _sget(csr_v, cb + BLK)
                    win_fetch((v0n // 8) * 8, 1 - par)

                def issue(c, gbuf):
                    @pl.when(c >= WIN)
                    def _():
                        pltpu.sync_copy(
                            sc_hbm.at[pl.ds(
                                pl.multiple_of(ea + c * BLK, 8), BLK)],
                            s_ws[par].at[pl.ds((c % WIN) * BLK, BLK)])
                    pltpu.async_copy(y_hbm.at[pl.ds(ea + c * BLK, BLK)],
                                     rows_v.at[gbuf], semg[gbuf])

                @pl.when(nch > 0)
                def _():
                    issue(0, 0)

                @pl.when(b > 0)
                def _():
                    pltpu.make_async_copy(
                        out_buf.at[pl.ds(0, BLK)],
                        out_hbm.at[pl.ds(pt0 + cb - BLK, BLK)], semo).wait()

                npair = (nch + 1) // 2

                def pairbody(g, carry):
                    for gbuf in range(2):
                        c = 2 * g + gbuf

                        @pl.when(c + 1 < nch)
                        def _():
                            issue(c + 1, 1 - gbuf)

                        @pl.when(c < nch)
                        def _():
                            pltpu.make_async_copy(
                                y_hbm.at[pl.ds(ea, BLK)], rows_v.at[gbuf],
                                semg[gbuf]).wait()
                        base = ea + c * BLK
                        woff = (c % WIN) * BLK
                        for g16 in range(BLK // 16):
                            iv = base + g16 * 16 + lanes
                            seg_v[pl.ds(g16 * 16, 16)] = _segids16(
                                csr_v, cb, iv, v0, v1)

                        def group(gi, st):
                            segv = seg_v[pl.ds(gi * 16, 16)]
                            sv16 = s_ws[par][pl.ds(woff + gi * 16, 16)]
                            for u in range(16):
                                p_ = st[0]
                                m_, d_ = st[1], st[2]
                                acc = st[3:]
                                p_new = segv[u]
                                ch = p_new != p_

                                @pl.when(ch)
                                def _():
                                    inv = 1.0 / (d_ + eps)
                                    for kk in range(KSUB):
                                        out_buf[p_, pl.ds(kk * 16, 16)] = \
                                            acc[kk] * inv

                                m2 = jnp.where(ch, negv, m_)
                                d2 = jnp.where(ch, zero, d_)
                                s_b = jnp.full((16,), sv16[u], jnp.float32)
                                m_new = jnp.maximum(m2, s_b)
                                scale = jnp.exp(m2 - m_new)
                                e = jnp.exp(s_b - m_new)
                                d_new = d2 * scale + e
                                out = []
                                for kk in range(KSUB):
                                    a = jnp.where(ch, zero, acc[kk])
                                    a = a * scale + e * rows_v[
                                        gbuf, gi * 16 + u, pl.ds(kk * 16, 16)]
                                    out.append(a)
                                st = (p_new, m_new, d_new) + tuple(out)
                            return st

                        carry = lax.fori_loop(0, BLK // 16, group, carry)
                    return carry

                init = (jnp.int32(0), negv, zero) + (zero,) * KSUB
                st = lax.fori_loop(0, npair, pairbody, init)
                p_fin = st[0]
                d_fin = st[2]
                acc_fin = st[3:]
                inv_fin = 1.0 / (d_fin + eps)
                for kk in range(KSUB):
                    out_buf[p_fin, pl.ds(kk * 16, 16)] = acc_fin[kk] * inv_fin

                def fix(r, _2):
                    @pl.when(_sget(csr_v, cb + r + 1) == _sget(csr_v, cb + r))
                    def _():
                        for kk in range(KSUB):
                            out_buf[r, pl.ds(kk * 16, 16)] = zero
                    return 0

                lax.fori_loop(0, BLK, fix, 0)
                pltpu.async_copy(out_buf.at[pl.ds(0, BLK)],
                                 out_hbm.at[pl.ds(pt0 + cb, BLK)], semo)
            return 0

        lax.fori_loop(0, nblk // 2, blockpair, 0)
        pltpu.make_async_copy(
            out_buf.at[pl.ds(0, BLK)],
            out_hbm.at[pl.ds(pt0 + (nblk - 1) * BLK, BLK)], semo).wait()

    return k(y, scores_pad, csr_pad)


def kernel(x_3d, mod_x, feature_map_indexing, atomic_csr, view_csr, x_map,
           W_fuse, b_fuse, w_att):
    n_view = x_map.shape[0]
    n_pts = x_3d.shape[0]

    scores = _scores_tc(x_map, w_att)[:, 0]
    scores_pad = jnp.pad(scores, (0, 2048))

    y = _atomic_pool_sc(mod_x, feature_map_indexing, atomic_csr, n_view)
    x_pool = _view_pool_sc(y, scores_pad, view_csr, n_pts)

    fused = _fuse_tc(x_3d, x_pool, W_fuse, b_fuse)
    x_seen = view_csr[1:] > view_csr[:-1]
    return fused, x_seen
